# baseline (speedup 1.0000x reference)
import jax
import jax.numpy as jnp
from jax.experimental import pallas as pl
from jax.experimental.pallas import tpu as pltpu

_BK = 256


def _gcnn_kernel(adj_ref, x_hbm, w_hbm, b_hbm, out_hbm,
                 ab_ref, cs_ref, xv, wv, bv, outv,
                 sem_x, sem_w, sem_b, sem_o):
    k = pl.program_id(0)
    num_blocks = pl.num_programs(0) - 1

    @pl.when(k == 0)
    def _start():
        cs_ref[...] = jnp.zeros_like(cs_ref)
        pltpu.make_async_copy(x_hbm, xv, sem_x).start()
        pltpu.make_async_copy(w_hbm, wv, sem_w).start()
        pltpu.make_async_copy(b_hbm, bv, sem_b).start()

    @pl.when(k < num_blocks)
    def _accumulate():
        blk = adj_ref[...]
        cs_ref[...] += jnp.sum(blk, axis=0, keepdims=True)
        ab_ref[pl.ds(k * _BK, _BK), :] = blk.astype(jnp.bfloat16)

    @pl.when(k == num_blocks)
    def _finalize():
        pltpu.make_async_copy(x_hbm, xv, sem_x).wait()
        pltpu.make_async_copy(w_hbm, wv, sem_w).wait()
        pltpu.make_async_copy(b_hbm, bv, sem_b).wait()
        d = jax.lax.rsqrt(cs_ref[...].astype(jnp.float32) + 1.0)
        dc = d.reshape(-1, 1)
        xw = jnp.dot(xv[...], wv[...], preferred_element_type=jnp.float32)
        y = xw * dc
        zt = jnp.dot(y.astype(jnp.bfloat16).T, ab_ref[...],
                     preferred_element_type=jnp.float32)
        outv[...] = (zt.T + y) * dc + bv[...]
        out_copy = pltpu.make_async_copy(outv, out_hbm, sem_o)
        out_copy.start()
        out_copy.wait()


def kernel(batch_inputs, batch_graph, W, b):
    n, f = batch_inputs.shape
    fo = W.shape[1]
    num_blocks = n // _BK
    return pl.pallas_call(
        _gcnn_kernel,
        grid=(num_blocks + 1,),
        in_specs=[
            pl.BlockSpec((_BK, n), lambda k: (jnp.minimum(k, num_blocks - 1), 0)),
            pl.BlockSpec(memory_space=pl.ANY),
            pl.BlockSpec(memory_space=pl.ANY),
            pl.BlockSpec(memory_space=pl.ANY),
        ],
        out_specs=pl.BlockSpec(memory_space=pl.ANY),
        scratch_shapes=[
            pltpu.VMEM((n, n), jnp.bfloat16),
            pltpu.VMEM((1, n), jnp.int32),
            pltpu.VMEM((n, f), jnp.float32),
            pltpu.VMEM((f, fo), jnp.float32),
            pltpu.VMEM((1, fo), jnp.float32),
            pltpu.VMEM((n, fo), jnp.float32),
            pltpu.SemaphoreType.DMA,
            pltpu.SemaphoreType.DMA,
            pltpu.SemaphoreType.DMA,
            pltpu.SemaphoreType.DMA,
        ],
        out_shape=jax.ShapeDtypeStruct((n, fo), batch_inputs.dtype),
    )(batch_graph, batch_inputs, W, b.reshape(1, -1))


# speedup vs baseline: 1.1697x; 1.1697x over previous
import jax
import jax.numpy as jnp
from jax.experimental import pallas as pl


def _gcnn_kernel(adj_ref, x_ref, w_ref, b_ref, out_ref):
    ai = adj_ref[...]
    deg = jnp.sum(ai, axis=0, keepdims=True)
    d = jax.lax.rsqrt(deg.astype(jnp.float32) + 1.0)
    dc = d.reshape(-1, 1)
    xw = jnp.dot(x_ref[...], w_ref[...], preferred_element_type=jnp.float32)
    y = xw * dc
    zt = jax.lax.dot_general(y.astype(jnp.bfloat16), ai.astype(jnp.bfloat16),
                             (((0,), (0,)), ((), ())),
                             preferred_element_type=jnp.float32)
    out_ref[...] = (zt.T + y) * dc + b_ref[...]


def kernel(batch_inputs, batch_graph, W, b):
    n, f = batch_inputs.shape
    return pl.pallas_call(
        _gcnn_kernel,
        out_shape=jax.ShapeDtypeStruct((n, W.shape[1]), batch_inputs.dtype),
    )(batch_graph, batch_inputs, W, b.reshape(1, -1))
